# baseline (speedup 1.0000x reference)
import functools

import jax
import jax.numpy as jnp
from jax import lax
from jax.experimental import pallas as pl
from jax.experimental.pallas import tpu as pltpu
from jax.experimental.pallas import tpu_sc as plsc


@functools.lru_cache
def _make_sc_gather_p(B, C):
    info = plsc.get_sparse_core_info()
    nc, ns, nl = info.num_cores, info.num_subcores, info.num_lanes
    nw = nc * ns
    cols_per_w = B // nw

    @functools.partial(
        pl.kernel,
        mesh=plsc.VectorSubcoreMesh(core_axis_name="c", subcore_axis_name="s"),
        out_type=jax.ShapeDtypeStruct((B,), jnp.float32),
        scratch_types=[
            pltpu.VMEM((C, cols_per_w), jnp.float32),
            pltpu.VMEM((cols_per_w,), jnp.int32),
            pltpu.VMEM((cols_per_w,), jnp.float32),
        ],
        compiler_params=pltpu.CompilerParams(
            use_tc_tiling_on_sc=True, needs_layout_passes=False
        ),
    )
    def sc_gather(phi_hbm, labels_hbm, out_hbm, buf, lab_v, p_v):
        wid = lax.axis_index("s") * nc + lax.axis_index("c")
        base = wid * cols_per_w
        pltpu.sync_copy(labels_hbm.at[pl.ds(base, cols_per_w)], lab_v)
        pltpu.sync_copy(phi_hbm.at[:, pl.ds(base, cols_per_w)], buf)
        for g in range(cols_per_w // nl):
            l = lab_v[pl.ds(g * nl, nl)]
            cvec = g * nl + lax.iota(jnp.int32, nl)
            p_v[pl.ds(g * nl, nl)] = plsc.load_gather(buf, [l, cvec])
        pltpu.sync_copy(p_v, out_hbm.at[pl.ds(base, cols_per_w)])

    return sc_gather


@functools.lru_cache
def _make_tc_stats(B, C, bs):
    nblk = B // bs

    def body(cos_ref, lbl_ref, m_ref, s_ref, c_ref):
        cos = cos_ref[...]
        lbl = lbl_ref[...]
        mask = lax.broadcasted_iota(jnp.int32, (C, bs), 0) == lbl
        m = jnp.max(cos, axis=0, keepdims=True)
        m_ref[...] = m
        s_ref[...] = jnp.sum(jnp.exp(cos - m), axis=0, keepdims=True)
        c_ref[...] = jnp.sum(jnp.where(mask, cos, 0.0), axis=0, keepdims=True)

    return pl.pallas_call(
        body,
        grid=(nblk,),
        in_specs=[
            pl.BlockSpec((C, bs), lambda i: (0, i)),
            pl.BlockSpec((1, bs), lambda i: (0, i)),
        ],
        out_specs=[pl.BlockSpec((1, bs), lambda i: (0, i))] * 3,
        out_shape=[jax.ShapeDtypeStruct((1, B), jnp.float32)] * 3,
    )


@functools.lru_cache
def _make_tc_finalize(B):
    def body(m_ref, s_ref, c_ref, p_ref, out_ref):
        m = m_ref[...]
        s = s_ref[...]
        c = c_ref[...]
        p = p_ref[...].reshape(1, B)
        m2 = jnp.maximum(m, p)
        s2 = s * jnp.exp(m - m2) - jnp.exp(c - m2) + jnp.exp(p - m2)
        nll = m2 + jnp.log(s2) - p
        out_ref[...] = jnp.sum(nll, keepdims=True) / B

    return pl.pallas_call(
        body, out_shape=jax.ShapeDtypeStruct((1, 1), jnp.float32)
    )


def kernel(cos_theta, phi_theta, labels):
    B, C = cos_theta.shape
    p = _make_sc_gather_p(B, C)(phi_theta.T, labels)
    m, s, c = _make_tc_stats(B, C, 1024)(cos_theta.T, labels.reshape(1, B))
    out = _make_tc_finalize(B)(m, s, c, p)
    return out[0, 0]


# speedup vs baseline: 2.2197x; 2.2197x over previous
import functools

import jax
import jax.numpy as jnp
from jax import lax
from jax.experimental import pallas as pl


@functools.lru_cache
def _make_tc_loss(B, C, bs):
    nblk = B // bs

    def body(cos_ref, phi_ref, lbl_ref, out_ref):
        i = pl.program_id(0)
        cos = cos_ref[...]
        phi = phi_ref[...]
        lbl = lbl_ref[...]
        mask = lax.broadcasted_iota(jnp.int32, (C, bs), 0) == lbl
        c = jnp.sum(jnp.where(mask, cos, 0.0), axis=0, keepdims=True)
        p = jnp.sum(jnp.where(mask, phi, 0.0), axis=0, keepdims=True)
        m = jnp.maximum(jnp.max(cos, axis=0, keepdims=True), p)
        s = (
            jnp.sum(jnp.exp(cos - m), axis=0, keepdims=True)
            - jnp.exp(c - m)
            + jnp.exp(p - m)
        )
        part = jnp.sum(m + jnp.log(s) - p, keepdims=True)

        @pl.when(i == 0)
        def _init():
            out_ref[...] = jnp.zeros_like(out_ref)

        out_ref[...] += part

        @pl.when(i == nblk - 1)
        def _final():
            out_ref[...] = out_ref[...] / B

    return pl.pallas_call(
        body,
        grid=(nblk,),
        in_specs=[
            pl.BlockSpec((C, bs), lambda i: (0, i)),
            pl.BlockSpec((C, bs), lambda i: (0, i)),
            pl.BlockSpec((1, bs), lambda i: (0, i)),
        ],
        out_specs=pl.BlockSpec((1, 1), lambda i: (0, 0)),
        out_shape=jax.ShapeDtypeStruct((1, 1), jnp.float32),
    )


def kernel(cos_theta, phi_theta, labels):
    B, C = cos_theta.shape
    out = _make_tc_loss(B, C, 1024)(
        cos_theta.T, phi_theta.T, labels.reshape(1, B)
    )
    return out[0, 0]


# speedup vs baseline: 2.2746x; 1.0247x over previous
import functools

import jax
import jax.numpy as jnp
from jax import lax
from jax.experimental import pallas as pl


@functools.lru_cache
def _make_tc_loss(B, C, bs):
    nblk = B // bs

    def body(cos_ref, phi_ref, lbl_ref, out_ref):
        i = pl.program_id(0)
        cos = cos_ref[...]
        phi = phi_ref[...]
        lbl = lbl_ref[...]
        mask = lax.broadcasted_iota(jnp.int32, (C, bs), 0) == lbl
        val = jnp.where(mask, phi, cos)
        p = jnp.sum(jnp.where(mask, phi, 0.0), axis=0, keepdims=True)
        m = jnp.max(val, axis=0, keepdims=True)
        s = jnp.sum(jnp.exp(val - m), axis=0, keepdims=True)
        part = jnp.sum(m + jnp.log(s) - p, keepdims=True)

        @pl.when(i == 0)
        def _init():
            out_ref[...] = jnp.zeros_like(out_ref)

        out_ref[...] += part

        @pl.when(i == nblk - 1)
        def _final():
            out_ref[...] = out_ref[...] / B

    return pl.pallas_call(
        body,
        grid=(nblk,),
        in_specs=[
            pl.BlockSpec((C, bs), lambda i: (0, i)),
            pl.BlockSpec((C, bs), lambda i: (0, i)),
            pl.BlockSpec((1, bs), lambda i: (0, i)),
        ],
        out_specs=pl.BlockSpec((1, 1), lambda i: (0, 0)),
        out_shape=jax.ShapeDtypeStruct((1, 1), jnp.float32),
    )


def kernel(cos_theta, phi_theta, labels):
    B, C = cos_theta.shape
    out = _make_tc_loss(B, C, 1024)(
        cos_theta.T, phi_theta.T, labels.reshape(1, B)
    )
    return out[0, 0]
